# baseline (speedup 1.0000x reference)
import functools

import jax
import jax.numpy as jnp
from jax import lax
from jax.experimental import pallas as pl
from jax.experimental.pallas import tpu as pltpu
from jax.experimental.pallas import tpu_sc as plsc

N = 4096
K = 128
R2 = 0.25
CH = 64
EPS = 1e-8

BI_M = 256
JC = 128
NC = N // JC
BI_A = 128
BI_B = 256

NCORES = 2
NSUB = 16
NW = NCORES * NSUB
ROWS_PW = N // NW
RB = 8


def _leaky(x):
    return jnp.where(x >= 0, x, 0.01 * x)


def _cumsum_lanes(v):
    iota = lax.broadcasted_iota(jnp.int32, v.shape, 1)
    cs = v
    s = 1
    while s < v.shape[1]:
        r = pltpu.roll(cs, s, 1)
        cs = cs + jnp.where(iota >= s, r, 0.0)
        s *= 2
    return cs


def _kernel_m(pct_ref, pcb_ref, w_ref, v_ref, cnt_ref, j0_ref):
    xi = pcb_ref[:, 0:1]
    yi = pcb_ref[:, 1:2]
    zi = pcb_ref[:, 2:3]

    starts = []
    run = jnp.zeros((BI_M, 1), jnp.float32)
    for c in range(NC):
        sl = pl.ds(c * JC, JC)
        rx = pct_ref[0:1, sl] - xi
        ry = pct_ref[1:2, sl] - yi
        rz = pct_ref[2:3, sl] - zi
        d2 = (rx * rx + ry * ry) + rz * rz
        vf = (d2 <= R2).astype(jnp.float32)
        starts.append(run)
        run = run + jnp.sum(vf, axis=1, keepdims=True)
    total = run
    cntc = jnp.minimum(total, float(K))
    pad = K - cntc
    cnt_ref[:, :] = cntc
    starts_mat = jnp.concatenate(starts, axis=1)

    def body(c, j0acc):
        sl = pl.ds(c * JC, JC)
        rx = pct_ref[0:1, sl] - xi
        ry = pct_ref[1:2, sl] - yi
        rz = pct_ref[2:3, sl] - zi
        d2 = (rx * rx + ry * ry) + rz * rz
        vf = (d2 <= R2).astype(jnp.float32)
        col = lax.broadcasted_iota(jnp.int32, (BI_M, NC), 1)
        start = jnp.sum(jnp.where(col == c, starts_mat, 0.0), axis=1,
                        keepdims=True)
        ex = _cumsum_lanes(vf) - vf + start
        sel = vf * (ex < float(K)).astype(jnp.float32)
        firstm = vf * (ex == 0.0).astype(jnp.float32)
        w_ref[:, sl] = sel + firstm * pad
        v_ref[:, sl] = sel * (ex + 1.0)
        jvals = (lax.broadcasted_iota(jnp.int32, (BI_M, JC), 1)
                 + JC * c).astype(jnp.float32)
        return j0acc + jnp.sum(firstm * jvals, axis=1, keepdims=True)

    j0 = lax.fori_loop(0, NC, body, jnp.zeros((BI_M, 1), jnp.float32))
    j0_ref[:, :] = j0


_SC_MESH = plsc.VectorSubcoreMesh(core_axis_name="c", subcore_axis_name="s")


@functools.partial(
    pl.kernel, mesh=_SC_MESH,
    compiler_params=pltpu.CompilerParams(needs_layout_passes=False),
    out_type=[
        jax.ShapeDtypeStruct((N, K), jnp.int32),
        jax.ShapeDtypeStruct((N, 4, K), jnp.float32),
    ],
    scratch_types=[
        pltpu.VMEM((4, N), jnp.float32),
        pltpu.VMEM((RB, N), jnp.float32),
        pltpu.VMEM((RB, K + 16), jnp.int32),
        pltpu.VMEM((RB, 4, K), jnp.float32),
        pltpu.VMEM((ROWS_PW, 1), jnp.float32),
    ],
)
def _sc_compact(pc4_hbm, v_hbm, j0_hbm, idx_hbm, nbrpc_hbm,
                pc_v, row_v, idx_v, g_v, j0_v):
    wid = lax.axis_index("s") * NCORES + lax.axis_index("c")
    pltpu.sync_copy(pc4_hbm, pc_v)
    pltpu.sync_copy(j0_hbm.at[pl.ds(wid * ROWS_PW, ROWS_PW)], j0_v)
    lanes = lax.iota(jnp.int32, 16)

    def batch_body(b, carry):
        i0 = wid * ROWS_PW + b * RB
        pltpu.sync_copy(v_hbm.at[pl.ds(i0, RB)], row_v)
        for rr in range(RB):
            rvec = jnp.full((16,), rr, jnp.int32) + b * RB
            j0 = plsc.load_gather(
                j0_v, [rvec, jnp.zeros((16,), jnp.int32)]).astype(jnp.int32)
            for g in range(K // 16):
                idx_v[rr, pl.ds(g * 16, 16)] = j0
            rsplat = jnp.full((16,), rr, jnp.int32)

            def step(s, c2):
                for u in range(4):
                    v = row_v[rr, pl.ds((s * 4 + u) * 16, 16)]
                    m = v > 0.5

                    def do_scatter(_):
                        pos = jnp.where(m, v.astype(jnp.int32) - 1, K)
                        jv = lanes + (s * 4 + u) * 16
                        plsc.store_scatter(idx_v, [rsplat, pos], jv, mask=m)
                        return 0

                    lax.cond(jnp.any(m), do_scatter, lambda _: 0, 0)
                return c2

            lax.fori_loop(0, N // 64, step, 0)
            for g in range(K // 16):
                ii = idx_v[rr, pl.ds(g * 16, 16)] & (N - 1)
                for c in range(4):
                    cvec = jnp.full((16,), c, jnp.int32)
                    g_v[rr, c, pl.ds(g * 16, 16)] = plsc.load_gather(
                        pc_v, [cvec, ii])
        pltpu.sync_copy(idx_v.at[:, pl.ds(0, K)], idx_hbm.at[pl.ds(i0, RB)])
        pltpu.sync_copy(g_v, nbrpc_hbm.at[pl.ds(i0, RB)])
        return carry

    lax.fori_loop(0, ROWS_PW // RB, batch_body, 0)


def _kernel_a2(nbr_ref, pcb_ref, w1s_ref, b1_ref, w2t_ref, b2_ref, feat_ref):
    xi = pcb_ref[:, 0:1]
    yi = pcb_ref[:, 1:2]
    zi = pcb_ref[:, 2:3]
    di = pcb_ref[:, 4:5]
    xj = nbr_ref[:, 0, :]
    yj = nbr_ref[:, 1, :]
    zj = nbr_ref[:, 2, :]
    dj = nbr_ref[:, 3, :]
    rx = xj - xi
    ry = yj - yi
    rz = zj - zi
    d2 = (rx * rx + ry * ry) + rz * rz
    d_rel = jnp.sqrt(d2)
    num1 = (xj * xi + yj * yi) + zj * zi
    num2 = (rx * xi + ry * yi) + rz * zi
    num3 = (rx * xj + ry * yj) + rz * zj
    cos1 = num1 / (dj * di + EPS)
    cos2 = num2 / (d_rel * di + EPS)
    cos3 = num3 / (d_rel * dj + EPS)
    f1 = jnp.broadcast_to(di, d2.shape)
    b1v = b1_ref[:, :].reshape(1, CH, 1)
    w1c = [w1s_ref[:, c:c + 1].reshape(1, CH, 1) for c in range(6)]
    h = (dj[:, None, :] * w1c[0] + f1[:, None, :] * w1c[1]
         + d_rel[:, None, :] * w1c[2] + cos1[:, None, :] * w1c[3]
         + cos2[:, None, :] * w1c[4] + cos3[:, None, :] * w1c[5]
         + b1v)
    h = _leaky(h)
    s0 = jnp.sum(h, axis=2)
    feat = lax.dot(s0 * (1.0 / K), w2t_ref[:, :],
                   precision=lax.Precision.HIGHEST)
    feat_ref[:, :] = feat + b2_ref[:, :]


def _kernel_l(feat_ref, w1t_ref, b1_ref, ell_ref):
    g = lax.dot(feat_ref[:, :], w1t_ref[:, :],
                precision=lax.Precision.HIGHEST) + b1_ref[:, :]
    ell_ref[:, :] = _leaky(g)


def _layer_body(ell_full, feat_blk, w_row, w2t, b2, wta, wtb, bt):
    s = lax.dot(w_row, ell_full, precision=lax.Precision.HIGHEST) * (1.0 / K)
    fm = lax.dot(s, w2t, precision=lax.Precision.HIGHEST) + b2
    return (lax.dot(feat_blk, wta, precision=lax.Precision.HIGHEST)
            + lax.dot(fm, wtb, precision=lax.Precision.HIGHEST) + bt)


def _kernel_b(ell_ref, featb_ref, w_ref, w2t_ref, b2_ref,
              wta_ref, wtb_ref, bt_ref, out_ref):
    out_ref[:, :] = _layer_body(
        ell_ref[:, :], featb_ref[:, :], w_ref[:, :],
        w2t_ref[:, :], b2_ref[:, :], wta_ref[:, :],
        wtb_ref[:, :], bt_ref[:, :])


def _kernel_c(ell_ref, featb_ref, w_ref, w2t_ref, b2_ref,
              wta_ref, wtb_ref, bt_ref, wot_ref, bo_ref, out_ref):
    nf = _layer_body(
        ell_ref[:, :], featb_ref[:, :], w_ref[:, :],
        w2t_ref[:, :], b2_ref[:, :], wta_ref[:, :],
        wtb_ref[:, :], bt_ref[:, :])
    out_ref[:, :] = (lax.dot(nf, wot_ref[:, :],
                             precision=lax.Precision.HIGHEST)
                     + bo_ref[:, :])


def kernel(pc, w_in1, b_in1, w_in2, b_in2, w_l0_1, b_l0_1, w_l0_2, b_l0_2,
           w_t0, b_t0, w_l1_1, b_l1_1, w_l1_2, b_l1_2, w_t1, b_t1,
           w_out, b_out):
    p = pc[0]
    n2 = jnp.sum(p * p, axis=-1)
    dn = jnp.sqrt(n2)
    pcb = jnp.concatenate(
        [p, n2[:, None], dn[:, None], jnp.zeros((N, 3), jnp.float32)], axis=1)
    pct = pcb.T
    pc4 = jnp.concatenate([p, dn[:, None]], axis=1).T

    w_mat, v_mat, cntf, j0f = pl.pallas_call(
        _kernel_m,
        grid=(N // BI_M,),
        in_specs=[
            pl.BlockSpec((8, N), lambda i: (0, 0)),
            pl.BlockSpec((BI_M, 8), lambda i: (i, 0)),
        ],
        out_specs=[
            pl.BlockSpec((BI_M, N), lambda i: (i, 0)),
            pl.BlockSpec((BI_M, N), lambda i: (i, 0)),
            pl.BlockSpec((BI_M, 1), lambda i: (i, 0)),
            pl.BlockSpec((BI_M, 1), lambda i: (i, 0)),
        ],
        out_shape=[
            jax.ShapeDtypeStruct((N, N), jnp.float32),
            jax.ShapeDtypeStruct((N, N), jnp.float32),
            jax.ShapeDtypeStruct((N, 1), jnp.float32),
            jax.ShapeDtypeStruct((N, 1), jnp.float32),
        ],
    )(pct, pcb)
    del cntf

    nbr_idx, nbrpc = _sc_compact(pc4, v_mat, j0f)
    del nbr_idx

    feat = pl.pallas_call(
        _kernel_a2,
        grid=(N // BI_A,),
        in_specs=[
            pl.BlockSpec((BI_A, 4, K), lambda i: (i, 0, 0)),
            pl.BlockSpec((BI_A, 8), lambda i: (i, 0)),
            pl.BlockSpec((CH, 6), lambda i: (0, 0)),
            pl.BlockSpec((1, CH), lambda i: (0, 0)),
            pl.BlockSpec((CH, CH), lambda i: (0, 0)),
            pl.BlockSpec((1, CH), lambda i: (0, 0)),
        ],
        out_specs=pl.BlockSpec((BI_A, CH), lambda i: (i, 0)),
        out_shape=jax.ShapeDtypeStruct((N, CH), jnp.float32),
    )(nbrpc, pcb, w_in1, b_in1[None, :], w_in2.T, b_in2[None, :])

    def run_layer(feat_in, w1, b1, w2, b2, wt, bt, final):
        wta = wt[:, :CH].T
        wtb = wt[:, CH:].T
        ell = pl.pallas_call(
            _kernel_l,
            grid=(1,),
            in_specs=[
                pl.BlockSpec((N, CH), lambda i: (0, 0)),
                pl.BlockSpec((CH, CH), lambda i: (0, 0)),
                pl.BlockSpec((1, CH), lambda i: (0, 0)),
            ],
            out_specs=pl.BlockSpec((N, CH), lambda i: (0, 0)),
            out_shape=jax.ShapeDtypeStruct((N, CH), jnp.float32),
        )(feat_in, w1.T, b1[None, :])
        common_specs = [
            pl.BlockSpec((N, CH), lambda i: (0, 0)),
            pl.BlockSpec((BI_B, CH), lambda i: (i, 0)),
            pl.BlockSpec((BI_B, N), lambda i: (i, 0)),
            pl.BlockSpec((CH, CH), lambda i: (0, 0)),
            pl.BlockSpec((1, CH), lambda i: (0, 0)),
            pl.BlockSpec((CH, CH), lambda i: (0, 0)),
            pl.BlockSpec((CH, CH), lambda i: (0, 0)),
            pl.BlockSpec((1, CH), lambda i: (0, 0)),
        ]
        args = [ell, feat_in, w_mat, w2.T, b2[None, :],
                wta, wtb, bt[None, :]]
        if final:
            specs = common_specs + [
                pl.BlockSpec((CH, 2 * CH), lambda i: (0, 0)),
                pl.BlockSpec((1, 2 * CH), lambda i: (0, 0)),
            ]
            return pl.pallas_call(
                _kernel_c,
                grid=(N // BI_B,),
                in_specs=specs,
                out_specs=pl.BlockSpec((BI_B, 2 * CH), lambda i: (i, 0)),
                out_shape=jax.ShapeDtypeStruct((N, 2 * CH), jnp.float32),
            )(*args, w_out.T, b_out[None, :])
        return pl.pallas_call(
            _kernel_b,
            grid=(N // BI_B,),
            in_specs=common_specs,
            out_specs=pl.BlockSpec((BI_B, CH), lambda i: (i, 0)),
            out_shape=jax.ShapeDtypeStruct((N, CH), jnp.float32),
        )(*args)

    feat = run_layer(feat, w_l0_1, b_l0_1, w_l0_2, b_l0_2, w_t0, b_t0, False)
    out = run_layer(feat, w_l1_1, b_l1_1, w_l1_2, b_l1_2, w_t1, b_t1, True)
    return out[None]


# speedup vs baseline: 1.9352x; 1.9352x over previous
import functools

import jax
import jax.numpy as jnp
from jax import lax
from jax.experimental import pallas as pl
from jax.experimental.pallas import tpu as pltpu
from jax.experimental.pallas import tpu_sc as plsc

N = 4096
K = 128
R2 = 0.25
CH = 64
EPS = 1e-8

BI_M = 256
JC = 128
NC = N // JC
BI_A = 128
BI_B = 256

NCORES = 2
NSUB = 16
NW = NCORES * NSUB
ROWS_PW = N // NW
RB = 8


def _leaky(x):
    return jnp.where(x >= 0, x, 0.01 * x)


def _cumsum_lanes(v):
    iota = lax.broadcasted_iota(jnp.int32, v.shape, 1)
    cs = v
    s = 1
    while s < v.shape[1]:
        r = pltpu.roll(cs, s, 1)
        cs = cs + jnp.where(iota >= s, r, 0.0)
        s *= 2
    return cs


def _kernel_m(pct_ref, pcb_ref, w_ref, v_ref, cnt_ref, j0_ref):
    xi = pcb_ref[:, 0:1]
    yi = pcb_ref[:, 1:2]
    zi = pcb_ref[:, 2:3]

    starts = []
    run = jnp.zeros((BI_M, 1), jnp.float32)
    for c in range(NC):
        sl = pl.ds(c * JC, JC)
        rx = pct_ref[0:1, sl] - xi
        ry = pct_ref[1:2, sl] - yi
        rz = pct_ref[2:3, sl] - zi
        d2 = (rx * rx + ry * ry) + rz * rz
        vf = (d2 <= R2).astype(jnp.float32)
        starts.append(run)
        run = run + jnp.sum(vf, axis=1, keepdims=True)
    total = run
    cntc = jnp.minimum(total, float(K))
    pad = K - cntc
    cnt_ref[:, :] = cntc
    starts_mat = jnp.concatenate(starts, axis=1)

    def body(c, j0acc):
        sl = pl.ds(c * JC, JC)
        rx = pct_ref[0:1, sl] - xi
        ry = pct_ref[1:2, sl] - yi
        rz = pct_ref[2:3, sl] - zi
        d2 = (rx * rx + ry * ry) + rz * rz
        vf = (d2 <= R2).astype(jnp.float32)
        col = lax.broadcasted_iota(jnp.int32, (BI_M, NC), 1)
        start = jnp.sum(jnp.where(col == c, starts_mat, 0.0), axis=1,
                        keepdims=True)
        ex = _cumsum_lanes(vf) - vf + start
        sel = vf * (ex < float(K)).astype(jnp.float32)
        firstm = vf * (ex == 0.0).astype(jnp.float32)
        w_ref[:, sl] = sel + firstm * pad
        v_ref[:, sl] = sel * (ex + 1.0)
        jvals = (lax.broadcasted_iota(jnp.int32, (BI_M, JC), 1)
                 + JC * c).astype(jnp.float32)
        return j0acc + jnp.sum(firstm * jvals, axis=1, keepdims=True)

    j0 = lax.fori_loop(0, NC, body, jnp.zeros((BI_M, 1), jnp.float32))
    j0_ref[:, :] = j0


_SC_MESH = plsc.VectorSubcoreMesh(core_axis_name="c", subcore_axis_name="s")


@functools.partial(
    pl.kernel, mesh=_SC_MESH,
    compiler_params=pltpu.CompilerParams(needs_layout_passes=False),
    out_type=[
        jax.ShapeDtypeStruct((N, K), jnp.int32),
        jax.ShapeDtypeStruct((N, 4, K), jnp.float32),
    ],
    scratch_types=[
        pltpu.VMEM((4, N), jnp.float32),
        pltpu.VMEM((RB, N), jnp.float32),
        pltpu.VMEM((RB, K + 16), jnp.int32),
        pltpu.VMEM((RB, 4, K), jnp.float32),
        pltpu.VMEM((ROWS_PW, 1), jnp.float32),
    ],
)
def _sc_compact(pc4_hbm, v_hbm, j0_hbm, idx_hbm, nbrpc_hbm,
                pc_v, row_v, idx_v, g_v, j0_v):
    wid = lax.axis_index("s") * NCORES + lax.axis_index("c")
    pltpu.sync_copy(pc4_hbm, pc_v)
    pltpu.sync_copy(j0_hbm.at[pl.ds(wid * ROWS_PW, ROWS_PW)], j0_v)
    lanes = lax.iota(jnp.int32, 16)

    def batch_body(b, carry):
        i0 = wid * ROWS_PW + b * RB
        pltpu.sync_copy(v_hbm.at[pl.ds(i0, RB)], row_v)
        for rr in range(RB):
            rvec = jnp.full((16,), rr, jnp.int32) + b * RB
            j0 = plsc.load_gather(
                j0_v, [rvec, jnp.zeros((16,), jnp.int32)]).astype(jnp.int32)
            for g in range(K // 16):
                idx_v[rr, pl.ds(g * 16, 16)] = j0
            rsplat = jnp.full((16,), rr, jnp.int32)

            @plsc.parallel_loop(0, N // 16, step=1, unroll=4)
            def _scan(s):
                v = row_v[rr, pl.ds(s * 16, 16)]
                m = v > 0.5
                pos = jnp.where(m, v.astype(jnp.int32) - 1, K)
                jv = lanes + s * 16
                plsc.store_scatter(idx_v, [rsplat, pos], jv, mask=m)
            for g in range(K // 16):
                ii = idx_v[rr, pl.ds(g * 16, 16)] & (N - 1)
                for c in range(4):
                    cvec = jnp.full((16,), c, jnp.int32)
                    g_v[rr, c, pl.ds(g * 16, 16)] = plsc.load_gather(
                        pc_v, [cvec, ii])
        pltpu.sync_copy(idx_v.at[:, pl.ds(0, K)], idx_hbm.at[pl.ds(i0, RB)])
        pltpu.sync_copy(g_v, nbrpc_hbm.at[pl.ds(i0, RB)])
        return carry

    lax.fori_loop(0, ROWS_PW // RB, batch_body, 0)


def _kernel_a2(nbr_ref, pcb_ref, w1s_ref, b1_ref, w2t_ref, b2_ref, feat_ref):
    xi = pcb_ref[:, 0:1]
    yi = pcb_ref[:, 1:2]
    zi = pcb_ref[:, 2:3]
    di = pcb_ref[:, 4:5]
    xj = nbr_ref[:, 0, :]
    yj = nbr_ref[:, 1, :]
    zj = nbr_ref[:, 2, :]
    dj = nbr_ref[:, 3, :]
    rx = xj - xi
    ry = yj - yi
    rz = zj - zi
    d2 = (rx * rx + ry * ry) + rz * rz
    d_rel = jnp.sqrt(d2)
    num1 = (xj * xi + yj * yi) + zj * zi
    num2 = (rx * xi + ry * yi) + rz * zi
    num3 = (rx * xj + ry * yj) + rz * zj
    cos1 = num1 / (dj * di + EPS)
    cos2 = num2 / (d_rel * di + EPS)
    cos3 = num3 / (d_rel * dj + EPS)
    f1 = jnp.broadcast_to(di, d2.shape)
    b1v = b1_ref[:, :].reshape(1, CH, 1)
    w1c = [w1s_ref[:, c:c + 1].reshape(1, CH, 1) for c in range(6)]
    h = (dj[:, None, :] * w1c[0] + f1[:, None, :] * w1c[1]
         + d_rel[:, None, :] * w1c[2] + cos1[:, None, :] * w1c[3]
         + cos2[:, None, :] * w1c[4] + cos3[:, None, :] * w1c[5]
         + b1v)
    h = _leaky(h)
    s0 = jnp.sum(h, axis=2)
    feat = lax.dot(s0 * (1.0 / K), w2t_ref[:, :],
                   precision=lax.Precision.HIGHEST)
    feat_ref[:, :] = feat + b2_ref[:, :]


def _kernel_l(feat_ref, w1t_ref, b1_ref, ell_ref):
    g = lax.dot(feat_ref[:, :], w1t_ref[:, :],
                precision=lax.Precision.HIGHEST) + b1_ref[:, :]
    ell_ref[:, :] = _leaky(g)


def _layer_body(ell_full, feat_blk, w_row, w2t, b2, wta, wtb, bt):
    s = lax.dot(w_row, ell_full, precision=lax.Precision.HIGHEST) * (1.0 / K)
    fm = lax.dot(s, w2t, precision=lax.Precision.HIGHEST) + b2
    return (lax.dot(feat_blk, wta, precision=lax.Precision.HIGHEST)
            + lax.dot(fm, wtb, precision=lax.Precision.HIGHEST) + bt)


def _kernel_b(ell_ref, featb_ref, w_ref, w2t_ref, b2_ref,
              wta_ref, wtb_ref, bt_ref, out_ref):
    out_ref[:, :] = _layer_body(
        ell_ref[:, :], featb_ref[:, :], w_ref[:, :],
        w2t_ref[:, :], b2_ref[:, :], wta_ref[:, :],
        wtb_ref[:, :], bt_ref[:, :])


def _kernel_c(ell_ref, featb_ref, w_ref, w2t_ref, b2_ref,
              wta_ref, wtb_ref, bt_ref, wot_ref, bo_ref, out_ref):
    nf = _layer_body(
        ell_ref[:, :], featb_ref[:, :], w_ref[:, :],
        w2t_ref[:, :], b2_ref[:, :], wta_ref[:, :],
        wtb_ref[:, :], bt_ref[:, :])
    out_ref[:, :] = (lax.dot(nf, wot_ref[:, :],
                             precision=lax.Precision.HIGHEST)
                     + bo_ref[:, :])


def kernel(pc, w_in1, b_in1, w_in2, b_in2, w_l0_1, b_l0_1, w_l0_2, b_l0_2,
           w_t0, b_t0, w_l1_1, b_l1_1, w_l1_2, b_l1_2, w_t1, b_t1,
           w_out, b_out):
    p = pc[0]
    n2 = jnp.sum(p * p, axis=-1)
    dn = jnp.sqrt(n2)
    pcb = jnp.concatenate(
        [p, n2[:, None], dn[:, None], jnp.zeros((N, 3), jnp.float32)], axis=1)
    pct = pcb.T
    pc4 = jnp.concatenate([p, dn[:, None]], axis=1).T

    w_mat, v_mat, cntf, j0f = pl.pallas_call(
        _kernel_m,
        grid=(N // BI_M,),
        in_specs=[
            pl.BlockSpec((8, N), lambda i: (0, 0)),
            pl.BlockSpec((BI_M, 8), lambda i: (i, 0)),
        ],
        out_specs=[
            pl.BlockSpec((BI_M, N), lambda i: (i, 0)),
            pl.BlockSpec((BI_M, N), lambda i: (i, 0)),
            pl.BlockSpec((BI_M, 1), lambda i: (i, 0)),
            pl.BlockSpec((BI_M, 1), lambda i: (i, 0)),
        ],
        out_shape=[
            jax.ShapeDtypeStruct((N, N), jnp.float32),
            jax.ShapeDtypeStruct((N, N), jnp.float32),
            jax.ShapeDtypeStruct((N, 1), jnp.float32),
            jax.ShapeDtypeStruct((N, 1), jnp.float32),
        ],
    )(pct, pcb)
    del cntf

    nbr_idx, nbrpc = _sc_compact(pc4, v_mat, j0f)
    del nbr_idx

    feat = pl.pallas_call(
        _kernel_a2,
        grid=(N // BI_A,),
        in_specs=[
            pl.BlockSpec((BI_A, 4, K), lambda i: (i, 0, 0)),
            pl.BlockSpec((BI_A, 8), lambda i: (i, 0)),
            pl.BlockSpec((CH, 6), lambda i: (0, 0)),
            pl.BlockSpec((1, CH), lambda i: (0, 0)),
            pl.BlockSpec((CH, CH), lambda i: (0, 0)),
            pl.BlockSpec((1, CH), lambda i: (0, 0)),
        ],
        out_specs=pl.BlockSpec((BI_A, CH), lambda i: (i, 0)),
        out_shape=jax.ShapeDtypeStruct((N, CH), jnp.float32),
    )(nbrpc, pcb, w_in1, b_in1[None, :], w_in2.T, b_in2[None, :])

    def run_layer(feat_in, w1, b1, w2, b2, wt, bt, final):
        wta = wt[:, :CH].T
        wtb = wt[:, CH:].T
        ell = pl.pallas_call(
            _kernel_l,
            grid=(1,),
            in_specs=[
                pl.BlockSpec((N, CH), lambda i: (0, 0)),
                pl.BlockSpec((CH, CH), lambda i: (0, 0)),
                pl.BlockSpec((1, CH), lambda i: (0, 0)),
            ],
            out_specs=pl.BlockSpec((N, CH), lambda i: (0, 0)),
            out_shape=jax.ShapeDtypeStruct((N, CH), jnp.float32),
        )(feat_in, w1.T, b1[None, :])
        common_specs = [
            pl.BlockSpec((N, CH), lambda i: (0, 0)),
            pl.BlockSpec((BI_B, CH), lambda i: (i, 0)),
            pl.BlockSpec((BI_B, N), lambda i: (i, 0)),
            pl.BlockSpec((CH, CH), lambda i: (0, 0)),
            pl.BlockSpec((1, CH), lambda i: (0, 0)),
            pl.BlockSpec((CH, CH), lambda i: (0, 0)),
            pl.BlockSpec((CH, CH), lambda i: (0, 0)),
            pl.BlockSpec((1, CH), lambda i: (0, 0)),
        ]
        args = [ell, feat_in, w_mat, w2.T, b2[None, :],
                wta, wtb, bt[None, :]]
        if final:
            specs = common_specs + [
                pl.BlockSpec((CH, 2 * CH), lambda i: (0, 0)),
                pl.BlockSpec((1, 2 * CH), lambda i: (0, 0)),
            ]
            return pl.pallas_call(
                _kernel_c,
                grid=(N // BI_B,),
                in_specs=specs,
                out_specs=pl.BlockSpec((BI_B, 2 * CH), lambda i: (i, 0)),
                out_shape=jax.ShapeDtypeStruct((N, 2 * CH), jnp.float32),
            )(*args, w_out.T, b_out[None, :])
        return pl.pallas_call(
            _kernel_b,
            grid=(N // BI_B,),
            in_specs=common_specs,
            out_specs=pl.BlockSpec((BI_B, CH), lambda i: (i, 0)),
            out_shape=jax.ShapeDtypeStruct((N, CH), jnp.float32),
        )(*args)

    feat = run_layer(feat, w_l0_1, b_l0_1, w_l0_2, b_l0_2, w_t0, b_t0, False)
    out = run_layer(feat, w_l1_1, b_l1_1, w_l1_2, b_l1_2, w_t1, b_t1, True)
    return out[None]
